# baseline (speedup 1.0000x reference)
import functools

import jax
import jax.numpy as jnp
from jax import lax
from jax.experimental import pallas as pl
from jax.experimental.pallas import tpu as pltpu

N = 50000
NP = 51200
TILE = 2048
NTILES = NP // TILE
RB = NP // 128
TPB = TILE // 128
K = 500

_NEG = float("-inf")
_POS = float("inf")


def _count_ge_k_search(pred_count, lo0, hi0, k, iters):
    def body(_, carry):
        lo, hi = carry
        mid = lo + (hi - lo) // 2
        ok = pred_count(mid)
        lo2 = jnp.where(ok, lo, mid + 1)
        hi2 = jnp.where(ok, mid, hi)
        return lo2, hi2

    lo, hi = lax.fori_loop(0, iters, body, (lo0, hi0))
    return lo


def _fused_kernel(h_ref, w1t_ref, b1_ref, wat_ref, ba_ref, wbt_ref, bb_ref,
                  wct_ref, bc_ref, wcrt_ref, bcr_ref, pres_ref,
                  yprob_ref, yhat_ref, loss_ref,
                  sd0, sd1, sc0, sc1, se0, se1, se2):
    i = pl.program_id(0)

    @pl.when(i < NTILES)
    def _compute():
        x = h_ref[...]
        h1 = jnp.maximum(
            jnp.dot(x, w1t_ref[...], preferred_element_type=jnp.float32)
            + b1_ref[...], 0.0)
        a = jnp.tanh(
            jnp.dot(h1, wat_ref[...], preferred_element_type=jnp.float32)
            + ba_ref[...])
        g = jax.nn.sigmoid(
            jnp.dot(h1, wbt_ref[...], preferred_element_type=jnp.float32)
            + bb_ref[...])
        ag = a * g
        det = jnp.dot(ag, wct_ref[...], preferred_element_type=jnp.float32) \
            + bc_ref[...]
        cc = jnp.dot(h1, wcrt_ref[...], preferred_element_type=jnp.float32) \
            + bcr_ref[...]
        cls_logit = cc[:, 0:2]
        ref_logit = cc[:, 2:5]
        cm = jnp.max(cls_logit, axis=1, keepdims=True)
        ce_ = jnp.exp(cls_logit - cm)
        cls_s = ce_ / jnp.sum(ce_, axis=1, keepdims=True)
        rm = jnp.max(ref_logit, axis=1, keepdims=True)
        sh = ref_logit - rm
        lse = jnp.log(jnp.sum(jnp.exp(sh), axis=1, keepdims=True))
        ce3 = lse - sh

        base = i * TPB
        sl = pl.ds(base, TPB)
        sd0[sl, :] = det[:, 0].reshape(TPB, 128)
        sd1[sl, :] = det[:, 1].reshape(TPB, 128)
        sc0[sl, :] = cls_s[:, 0].reshape(TPB, 128)
        sc1[sl, :] = cls_s[:, 1].reshape(TPB, 128)
        se0[sl, :] = ce3[:, 0].reshape(TPB, 128)
        se1[sl, :] = ce3[:, 1].reshape(TPB, 128)
        se2[sl, :] = ce3[:, 2].reshape(TPB, 128)

    @pl.when(i == NTILES)
    def _finalize():
        ridx = lax.broadcasted_iota(jnp.int32, (RB, 128), 0)
        cidx = lax.broadcasted_iota(jnp.int32, (RB, 128), 1)
        flat = ridx * 128 + cidx
        valid = flat < N
        fvalid = valid.astype(jnp.float32)

        d0 = sd0[...]
        d1 = sd1[...]
        m0 = jnp.max(jnp.where(valid, d0, _NEG))
        m1 = jnp.max(jnp.where(valid, d1, _NEG))
        e0 = jnp.exp(d0 - m0)
        e1 = jnp.exp(d1 - m1)
        z0 = jnp.sum(jnp.where(valid, e0, 0.0))
        z1 = jnp.sum(jnp.where(valid, e1, 0.0))
        fs0 = sc0[...] * (e0 / z0)
        fs1 = sc1[...] * (e1 / z1)

        yp0 = jnp.clip(jnp.sum(jnp.where(valid, fs0, 0.0)),
                       1e-10, 1.0 - 1e-10)
        yp1 = jnp.clip(jnp.sum(jnp.where(valid, fs1, 0.0)),
                       1e-10, 1.0 - 1e-10)
        yhat = jnp.where(yp1 > yp0, 1, 0).astype(jnp.int32)

        def topk_group(s, ce, pres, largest, thresh_on_norm):
            smin = jnp.min(jnp.where(valid, s, _POS))
            smax = jnp.max(jnp.where(valid, s, _NEG))
            denom = smax - smin
            if thresh_on_norm:
                passes = ((s - smin) / denom) > 0.5
            else:
                passes = s < 0.5
            sm = jnp.where(valid, s, -1.0 if largest else _POS)
            bits = lax.bitcast_convert_type(sm, jnp.int32)

            if largest:
                def pred(x):
                    c = jnp.sum(jnp.where(bits > x, 1.0, 0.0))
                    return c < float(K)
            else:
                def pred(x):
                    c = jnp.sum(jnp.where(bits <= x, 1.0, 0.0))
                    return c >= float(K)

            tbits = _count_ge_k_search(pred, jnp.int32(0),
                                       jnp.int32(2**31 - 1), K, 31)
            if largest:
                strict = bits > tbits
            else:
                strict = bits < tbits
            n_strict = jnp.sum(jnp.where(strict, 1.0, 0.0))
            m = float(K) - n_strict
            tied = bits == tbits

            def ipred(x):
                c = jnp.sum(jnp.where(tied & (flat < x), 1.0, 0.0))
                return c >= m

            iend = _count_ge_k_search(ipred, jnp.int32(0), jnp.int32(NP),
                                      K, 17)
            tied_sel = tied & (flat < iend)

            t = lax.bitcast_convert_type(tbits, jnp.float32)
            if thresh_on_norm:
                t_pass = ((t - smin) / denom) > 0.5
            else:
                t_pass = t < 0.5

            sel1 = strict & passes
            part1 = jnp.sum(jnp.where(sel1, ce, 0.0))
            cnt1 = jnp.sum(jnp.where(sel1, 1.0, 0.0))
            part2 = jnp.where(t_pass, jnp.sum(jnp.where(tied_sel, ce, 0.0)),
                              0.0)
            cnt2 = jnp.where(t_pass, m, 0.0)
            return pres * (part1 + part2), pres * (cnt1 + cnt2)

        pres0 = pres_ref[0, 0]
        pres1 = pres_ref[0, 1]
        t0, c0 = topk_group(fs0, se0[...], pres0, True, True)
        t1, c1 = topk_group(fs1, se1[...], pres1, True, True)
        mean_s = (fs0 + fs1) / 2.0
        tn, cn = topk_group(mean_s, se2[...], 1.0, False, False)

        loss = (t0 + t1 + tn) / (c0 + c1 + cn)

        yprob_ref[...] = jnp.concatenate(
            [yp0.reshape(1, 1), yp1.reshape(1, 1)], axis=1)
        yhat_ref[...] = yhat.reshape(1, 1)
        loss_ref[...] = loss.reshape(1, 1)


@jax.jit
def kernel(h, W1, b1, Wa, ba, Wb, bb, Wc, bc, Wcls, bcls, Wref, bref, label):
    hp = jnp.pad(h, ((0, NP - N), (0, 0)))
    w1t = W1.T
    wat = Wa.T
    wbt = Wb.T
    wct = Wc.T
    wcrt = jnp.concatenate([Wcls, Wref], axis=0).T
    bcr = jnp.concatenate([bcls, bref], axis=0).reshape(1, 5)
    pres = jnp.stack([jnp.any(label == 0), jnp.any(label == 1)]) \
        .astype(jnp.float32).reshape(1, 2)

    full = lambda shape: pl.BlockSpec(shape, lambda i: (0,) * len(shape))
    grid_spec = pltpu.PrefetchScalarGridSpec(
        num_scalar_prefetch=0,
        grid=(NTILES + 1,),
        in_specs=[
            pl.BlockSpec((TILE, 1024), lambda i: (jnp.minimum(i, NTILES - 1), 0)),
            full((1024, 512)),
            full((1, 512)),
            full((512, 256)),
            full((1, 256)),
            full((512, 256)),
            full((1, 256)),
            full((256, 2)),
            full((1, 2)),
            full((512, 5)),
            full((1, 5)),
            full((1, 2)),
        ],
        out_specs=[
            full((1, 2)),
            full((1, 1)),
            full((1, 1)),
        ],
        scratch_shapes=[pltpu.VMEM((RB, 128), jnp.float32)] * 7,
    )
    yprob, yhat, loss = pl.pallas_call(
        _fused_kernel,
        grid_spec=grid_spec,
        out_shape=[
            jax.ShapeDtypeStruct((1, 2), jnp.float32),
            jax.ShapeDtypeStruct((1, 1), jnp.int32),
            jax.ShapeDtypeStruct((1, 1), jnp.float32),
        ],
        compiler_params=pltpu.CompilerParams(
            dimension_semantics=("arbitrary",),
        ),
    )(hp, w1t, b1.reshape(1, 512), wat, ba.reshape(1, 256), wbt,
      bb.reshape(1, 256), wct, bc.reshape(1, 2), wcrt, bcr, pres)
    return yprob[0], yhat[0, 0], loss[0, 0]


# speedup vs baseline: 3.3229x; 3.3229x over previous
import functools

import jax
import jax.numpy as jnp
from jax import lax
from jax.experimental import pallas as pl
from jax.experimental.pallas import tpu as pltpu

N = 50000
TILE = 2048
NTILES = -(-N // TILE)
NP = NTILES * TILE
RB = NP // 128
TPB = TILE // 128
K = 500

_NEG = float("-inf")
_POS = float("inf")
_PREC = lax.Precision.DEFAULT


def _dot(a, b):
    return jax.lax.dot_general(a, b, (((1,), (0,)), ((), ())),
                               preferred_element_type=jnp.float32,
                               precision=_PREC)


def _count_ge_k_search(pred_count, lo0, hi0, k, iters):
    def body(_, carry):
        lo, hi = carry
        mid = lo + (hi - lo) // 2
        ok = pred_count(mid)
        lo2 = jnp.where(ok, lo, mid + 1)
        hi2 = jnp.where(ok, mid, hi)
        return lo2, hi2

    lo, hi = lax.fori_loop(0, iters, body, (lo0, hi0))
    return lo


def _fused_kernel(h_ref, w1t_ref, b1_ref, wat_ref, ba_ref,
                  wct_ref, bc_ref, pres_ref,
                  yprob_ref, yhat_ref, loss_ref,
                  h1b, sd0, sd1, sc0, sc1, se0, se1, se2):
    i = pl.program_id(0)

    pbase = jnp.maximum(i - 1, 0)
    h1p = h1b[pbase % 2]
    abc = _dot(h1p, wat_ref[...]) + ba_ref[...]
    a = jnp.tanh(abc[:, 0:256])
    g = jax.nn.sigmoid(abc[:, 256:512])
    cc = abc[:, 512:517]
    ag = a * g
    det = _dot(ag.astype(jnp.bfloat16), wct_ref[...]) + bc_ref[...]

    sl = pl.ds(pbase * TPB, TPB)
    sd0[sl, :] = det[:, 0].reshape(TPB, 128)
    sd1[sl, :] = det[:, 1].reshape(TPB, 128)
    sc0[sl, :] = cc[:, 0].reshape(TPB, 128)
    sc1[sl, :] = cc[:, 1].reshape(TPB, 128)
    se0[sl, :] = cc[:, 2].reshape(TPB, 128)
    se1[sl, :] = cc[:, 3].reshape(TPB, 128)
    se2[sl, :] = cc[:, 4].reshape(TPB, 128)

    x = h_ref[...].astype(jnp.bfloat16)
    h1 = jnp.maximum(_dot(x, w1t_ref[...]) + b1_ref[...], 0.0)
    h1b[i % 2] = h1.astype(jnp.bfloat16)

    @pl.when(i == NTILES)
    def _finalize():
        ridx = lax.broadcasted_iota(jnp.int32, (RB, 128), 0)
        cidx = lax.broadcasted_iota(jnp.int32, (RB, 128), 1)
        flat = ridx * 128 + cidx
        valid = flat < N
        fvalid = valid.astype(jnp.float32)

        d0 = sd0[...]
        d1 = sd1[...]
        cl0 = sc0[...]
        cl1 = sc1[...]
        cm = jnp.maximum(cl0, cl1)
        ce0_ = jnp.exp(cl0 - cm)
        ce1_ = jnp.exp(cl1 - cm)
        czs = ce0_ + ce1_
        cs0 = ce0_ / czs
        cs1 = ce1_ / czs
        r0 = se0[...]
        r1 = se1[...]
        r2 = se2[...]
        rm = jnp.maximum(jnp.maximum(r0, r1), r2)
        sh0 = r0 - rm
        sh1 = r1 - rm
        sh2 = r2 - rm
        lse = jnp.log(jnp.exp(sh0) + jnp.exp(sh1) + jnp.exp(sh2))
        ce_t0 = lse - sh0
        ce_t1 = lse - sh1
        ce_t2 = lse - sh2
        m0 = jnp.max(jnp.where(valid, d0, _NEG))
        m1 = jnp.max(jnp.where(valid, d1, _NEG))
        e0 = jnp.exp(d0 - m0)
        e1 = jnp.exp(d1 - m1)
        z0 = jnp.sum(jnp.where(valid, e0, 0.0))
        z1 = jnp.sum(jnp.where(valid, e1, 0.0))
        fs0 = cs0 * (e0 / z0)
        fs1 = cs1 * (e1 / z1)

        yp0 = jnp.clip(jnp.sum(jnp.where(valid, fs0, 0.0)),
                       1e-10, 1.0 - 1e-10)
        yp1 = jnp.clip(jnp.sum(jnp.where(valid, fs1, 0.0)),
                       1e-10, 1.0 - 1e-10)
        yhat = jnp.where(yp1 > yp0, 1, 0).astype(jnp.int32)

        def topk_group(s, ce, pres, largest, thresh_on_norm):
            smin = jnp.min(jnp.where(valid, s, _POS))
            smax = jnp.max(jnp.where(valid, s, _NEG))
            denom = smax - smin
            if thresh_on_norm:
                passes = ((s - smin) / denom) > 0.5
            else:
                passes = s < 0.5
            sm = jnp.where(valid, s, -1.0 if largest else _POS)
            bits = lax.bitcast_convert_type(sm, jnp.int32)

            if largest:
                def pred(x):
                    c = jnp.sum(jnp.where(bits > x, 1.0, 0.0))
                    return c < float(K)
            else:
                def pred(x):
                    c = jnp.sum(jnp.where(bits <= x, 1.0, 0.0))
                    return c >= float(K)

            tbits = _count_ge_k_search(pred, jnp.int32(0),
                                       jnp.int32(2**31 - 1), K, 31)
            if largest:
                strict = bits > tbits
            else:
                strict = bits < tbits
            n_strict = jnp.sum(jnp.where(strict, 1.0, 0.0))
            m = float(K) - n_strict
            tied = bits == tbits

            def ipred(x):
                c = jnp.sum(jnp.where(tied & (flat < x), 1.0, 0.0))
                return c >= m

            iend = _count_ge_k_search(ipred, jnp.int32(0), jnp.int32(NP),
                                      K, 17)
            tied_sel = tied & (flat < iend)

            t = lax.bitcast_convert_type(tbits, jnp.float32)
            if thresh_on_norm:
                t_pass = ((t - smin) / denom) > 0.5
            else:
                t_pass = t < 0.5

            sel1 = strict & passes
            part1 = jnp.sum(jnp.where(sel1, ce, 0.0))
            cnt1 = jnp.sum(jnp.where(sel1, 1.0, 0.0))
            part2 = jnp.where(t_pass, jnp.sum(jnp.where(tied_sel, ce, 0.0)),
                              0.0)
            cnt2 = jnp.where(t_pass, m, 0.0)
            return pres * (part1 + part2), pres * (cnt1 + cnt2)

        pres0 = pres_ref[0, 0]
        pres1 = pres_ref[0, 1]
        t0, c0 = topk_group(fs0, ce_t0, pres0, True, True)
        t1, c1 = topk_group(fs1, ce_t1, pres1, True, True)
        mean_s = (fs0 + fs1) / 2.0
        tn, cn = topk_group(mean_s, ce_t2, 1.0, False, False)

        loss = (t0 + t1 + tn) / (c0 + c1 + cn)

        yprob_ref[...] = jnp.concatenate(
            [yp0.reshape(1, 1), yp1.reshape(1, 1)], axis=1)
        yhat_ref[...] = yhat.reshape(1, 1)
        loss_ref[...] = loss.reshape(1, 1)


@jax.jit
def kernel(h, W1, b1, Wa, ba, Wb, bb, Wc, bc, Wcls, bcls, Wref, bref, label):
    hp = h
    w1t = W1.T.astype(jnp.bfloat16)
    wall = jnp.concatenate([Wa, Wb, Wcls, Wref], axis=0).T.astype(jnp.bfloat16)
    ball = jnp.concatenate([ba, bb, bcls, bref], axis=0).reshape(1, 517)
    wct = Wc.T.astype(jnp.bfloat16)
    pres = jnp.stack([jnp.any(label == 0), jnp.any(label == 1)]) \
        .astype(jnp.float32).reshape(1, 2)

    full = lambda shape: pl.BlockSpec(shape, lambda i: (0,) * len(shape))
    grid_spec = pltpu.PrefetchScalarGridSpec(
        num_scalar_prefetch=0,
        grid=(NTILES + 1,),
        in_specs=[
            pl.BlockSpec((TILE, 1024), lambda i: (jnp.minimum(i, NTILES - 1), 0)),
            full((1024, 512)),
            full((1, 512)),
            full((512, 517)),
            full((1, 517)),
            full((256, 2)),
            full((1, 2)),
            full((1, 2)),
        ],
        out_specs=[
            full((1, 2)),
            full((1, 1)),
            full((1, 1)),
        ],
        scratch_shapes=[pltpu.VMEM((2, TILE, 512), jnp.bfloat16)]
        + [pltpu.VMEM((RB, 128), jnp.float32)] * 7,
    )
    yprob, yhat, loss = pl.pallas_call(
        _fused_kernel,
        grid_spec=grid_spec,
        out_shape=[
            jax.ShapeDtypeStruct((1, 2), jnp.float32),
            jax.ShapeDtypeStruct((1, 1), jnp.int32),
            jax.ShapeDtypeStruct((1, 1), jnp.float32),
        ],
        compiler_params=pltpu.CompilerParams(
            dimension_semantics=("arbitrary",),
        ),
    )(hp, w1t, b1.reshape(1, 512), wall, ball, wct, bc.reshape(1, 2), pres)
    return yprob[0], yhat[0, 0], loss[0, 0]
